# baseline (speedup 1.0000x reference)
import functools
import math

import jax
import jax.numpy as jnp
from jax import lax
from jax.experimental import pallas as pl
from jax.experimental.pallas import tpu as pltpu
from jax.experimental.pallas import tpu_sc as plsc

LANES = 16


@functools.cache
def _build(B, F, D, V):
    info = plsc.get_sparse_core_info()
    NC, NS = info.num_cores, info.num_subcores
    NW = NC * NS
    N = B * F
    RW = N // NW
    assert N % NW == 0 and RW % F == 0
    R = 64 * F
    G = RW // R
    assert RW % R == 0 and R % LANES == 0 and D % LANES == 0

    mesh = plsc.VectorSubcoreMesh(core_axis_name="c", subcore_axis_name="s")

    @functools.partial(
        pl.kernel,
        out_type=jax.ShapeDtypeStruct((N, D), jnp.float32),
        mesh=mesh,
        compiler_params=pltpu.CompilerParams(use_tc_tiling_on_sc=False),
        scratch_types=[
            pltpu.VMEM((R,), jnp.int32),
            pltpu.VMEM((R,), jnp.int32),
            pltpu.VMEM((F, D), jnp.float32),
            pltpu.VMEM((R, D), jnp.float32),
            pltpu.SemaphoreType.DMA,
        ],
    )
    def k(x_hbm, offs_hbm, table_hbm, bias_hbm, out_hbm,
          idx_v, offs_v, bias_v, rows_v, sem):
        wid = lax.axis_index("s") * NC + lax.axis_index("c")
        base = wid * RW
        pltpu.sync_copy(offs_hbm, offs_v)
        pltpu.sync_copy(bias_hbm, bias_v)

        def chunk_body(g, carry):
            cbase = base + g * R
            pltpu.sync_copy(x_hbm.at[pl.ds(cbase, R)], idx_v)

            def add_off(v, c2):
                sl = pl.ds(v * LANES, LANES)
                idx_v[sl] = idx_v[sl] + offs_v[sl]
                return c2
            lax.fori_loop(0, R // LANES, add_off, 0)

            pltpu.async_copy(table_hbm.at[idx_v], rows_v, sem).wait()

            for f in range(F):
                brow = [bias_v[f, pl.ds(d * LANES, LANES)]
                        for d in range(D // LANES)]

                def badd(r, c2, f=f, brow=brow):
                    row = r * F + f
                    for d in range(D // LANES):
                        sl = pl.ds(d * LANES, LANES)
                        rows_v[row, sl] = rows_v[row, sl] + brow[d]
                    return c2
                lax.fori_loop(0, R // F, badd, 0)

            pltpu.sync_copy(rows_v, out_hbm.at[pl.ds(cbase, R)])
            return carry
        lax.fori_loop(0, G, chunk_body, 0)

    return k, R


def kernel(x, offsets, table, bias):
    B, F = x.shape
    V, D = table.shape
    k, R = _build(B, F, D, V)
    x_flat = x.reshape(B * F)
    offs_tiled = jnp.tile(offsets, R // F)
    out = k(x_flat, offs_tiled, table, bias)
    return out.reshape(B, F, D)
